# baseline (speedup 1.0000x reference)
import functools

import jax
import jax.numpy as jnp
import numpy as np
from jax import lax
from jax.experimental import pallas as pl
from jax.experimental.pallas import tpu as pltpu
from jax.experimental.pallas import tpu_sc as plsc

EPS = 1.0 + 1e-06

NC = 2
NS = 16
CH = 48


def _tc_prep(x, W_lower, W_upper, W_lin, A_lower, A_upper, blk):
    n, d = x.shape
    nblk = n // blk

    def body(x_ref, wl_ref, wu_ref, wn_ref, al_ref, au_ref,
             xml_ref, xmu_ref, wx_ref, aux_ref):
        xb = x_ref[...]
        xml = jnp.dot(xb, wl_ref[...], preferred_element_type=jnp.float32)
        xmu = jnp.dot(xb, wu_ref[...], preferred_element_type=jnp.float32)
        xml_ref[...] = xml
        xmu_ref[...] = xmu
        wx_ref[...] = jnp.dot(xb, wn_ref[...],
                              preferred_element_type=jnp.float32) * EPS
        aux_ref[...] = (jnp.dot(xml, al_ref[...],
                                preferred_element_type=jnp.float32) +
                        jnp.dot(xmu, au_ref[...],
                                preferred_element_type=jnp.float32))

    return pl.pallas_call(
        body,
        grid=(nblk,),
        in_specs=[
            pl.BlockSpec((blk, d), lambda i: (i, 0)),
            pl.BlockSpec((d, d), lambda i: (0, 0)),
            pl.BlockSpec((d, d), lambda i: (0, 0)),
            pl.BlockSpec((d, d), lambda i: (0, 0)),
            pl.BlockSpec((d, 8), lambda i: (0, 0)),
            pl.BlockSpec((d, 8), lambda i: (0, 0)),
        ],
        out_specs=[
            pl.BlockSpec((blk, d), lambda i: (i, 0)),
            pl.BlockSpec((blk, d), lambda i: (i, 0)),
            pl.BlockSpec((blk, d), lambda i: (i, 0)),
            pl.BlockSpec((blk, 8), lambda i: (i, 0)),
        ],
        out_shape=[
            jax.ShapeDtypeStruct((n, d), jnp.float32),
            jax.ShapeDtypeStruct((n, d), jnp.float32),
            jax.ShapeDtypeStruct((n, d), jnp.float32),
            jax.ShapeDtypeStruct((n, 8), jnp.float32),
        ],
    )(x, W_lower, W_upper, W_lin, A_lower, A_upper)


def _sc_message_pass(xm_packed, s_all, t_all, edata, n, d, nchunks):
    ept = nchunks * CH
    rpt = -(-n // (NS * 8)) * 8
    n_acc = NS * rpt
    nfull = rpt // CH
    rem = rpt - nfull * CH
    NB = 4

    mesh = plsc.VectorSubcoreMesh(core_axis_name="c", subcore_axis_name="s",
                                  num_cores=NC, num_subcores=NS)

    scratch = (
        [pltpu.VMEM((n,), jnp.float32)] * 2 +
        [pltpu.VMEM((3, CH), jnp.int32)] * NB +
        [pltpu.VMEM((CH,), jnp.int32)] * NB +
        [pltpu.VMEM((CH,), jnp.int32)] * NB +
        [pltpu.VMEM((CH,), jnp.float32)] * NB +
        [pltpu.VMEM((CH, 64), jnp.int32)] * NB +
        [pltpu.VMEM((CH, 128), jnp.float32)] * 2 +
        [pltpu.VMEM_SHARED((n_acc, 128), jnp.float32)] +
        [pltpu.SemaphoreType.DMA] * (2 * NB) +
        [pltpu.SemaphoreType.DMA] * 2
    )

    @functools.partial(
        pl.kernel,
        out_type=jax.ShapeDtypeStruct((NC * n_acc, d), jnp.float32),
        mesh=mesh,
        compiler_params=pltpu.CompilerParams(needs_layout_passes=False,
                                             use_tc_tiling_on_sc=False),
        scratch_types=scratch,
    )
    def k(xm_h, s_h, t_h, ed_h, out_h, s_v, t_v, *bufs):
        ebs = bufs[0:NB]
        tscs = bufs[NB:2 * NB]
        gscs = bufs[2 * NB:3 * NB]
        ews = bufs[3 * NB:4 * NB]
        rbuf = bufs[4 * NB:5 * NB]
        sbuf = bufs[5 * NB:5 * NB + 2]
        acc = bufs[5 * NB + 2]
        sems = bufs[5 * NB + 3:]
        sem_e = sems[0:NB]
        sem_r = sems[NB:2 * NB]
        sem_s = sems[2 * NB:2 * NB + 2]
        c = lax.axis_index("c")
        sid = lax.axis_index("s")
        base = sid * rpt

        pltpu.sync_copy(s_h.at[pl.ds(c * n, n)], s_v)
        pltpu.sync_copy(t_h.at[pl.ds(c * n, n)], t_v)

        def zero_rows(i, _):
            for kk in range(8):
                sbuf[0][i, pl.ds(kk * 16, 16)] = jnp.zeros((16,), jnp.float32)
            return 0
        lax.fori_loop(0, CH, zero_rows, 0)
        for j in range(nfull):
            pltpu.sync_copy(sbuf[0], acc.at[pl.ds(base + j * CH, CH)])
        if rem:
            pltpu.sync_copy(sbuf[0].at[pl.ds(0, rem)],
                            acc.at[pl.ds(base + nfull * CH, rem)])
        plsc.subcore_barrier()

        chunk_base = (c * NS + sid) * nchunks

        def issue_edata(j, b):
            pltpu.async_copy(ed_h.at[chunk_base + j], ebs[b], sem_e[b])

        def finish(bp, par, wait_scatter):
            rp = rbuf[bp]
            ewp = ews[bp]
            sp = sbuf[par]
            pltpu.make_async_copy(xm_h.at[gscs[bp]], rp, sem_r[bp]).wait()
            if wait_scatter is None:
                pltpu.make_async_copy(sp, acc.at[tscs[bp]], sem_s[par]).wait()
            else:
                @pl.when(wait_scatter)
                def _():
                    pltpu.make_async_copy(
                        sp, acc.at[tscs[bp]], sem_s[par]).wait()

            @plsc.parallel_loop(0, CH, unroll=2)
            def _(e2):
                ew16 = plsc.load_gather(ewp, [lax.broadcast(e2, (16,))])
                for g in range(4):
                    w = rp[e2, pl.ds(g * 16, 16)]
                    bf = plsc.bitcast(w, jnp.bfloat16)
                    lo, hi = plsc.unpack(
                        bf, format=plsc.PackFormat.INTERLEAVED,
                        preferred_element_type=jnp.float32)
                    sp[e2, pl.ds(g * 32, 16)] = lo * ew16
                    sp[e2, pl.ds(g * 32 + 16, 16)] = hi * ew16
            pltpu.async_copy(sp, acc.at[tscs[bp]], sem_s[par], add=True)

        def process(i, b):
            j = NB * i + b
            eb = ebs[b]
            tsc = tscs[b]
            gsc = gscs[b]
            ew_v = ews[b]
            pltpu.make_async_copy(ed_h.at[chunk_base], eb, sem_e[b]).wait()
            for i16 in range(CH // 16):
                sl = pl.ds(i16 * 16, 16)
                sv = eb[0, sl]
                tv = eb[1, sl]
                a = (plsc.load_gather(s_v, [sv - c * n]) +
                     plsc.load_gather(t_v, [tv]))
                att = jnp.where(a > 0.0, a, jnp.exp(a) - 1.0)
                ew_v[sl] = att * plsc.bitcast(eb[2, sl], jnp.float32)
                tsc[sl] = tv
                gsc[sl] = sv
            pltpu.async_copy(xm_h.at[gsc], rbuf[b], sem_r[b])
            @pl.when(j + NB < nchunks)
            def _():
                issue_edata(j + NB, b)
            if b == 0:
                @pl.when(i >= 1)
                def _():
                    finish(NB - 1, 1, None)
            elif b == 1:
                finish(0, 0, i >= 1)
            elif b == 2:
                finish(1, 1, i >= 1)
            else:
                finish(2, 0, None)

        for b in range(NB):
            issue_edata(b, b)

        def ring(i, _):
            for b in range(NB):
                process(i, b)
            return 0
        lax.fori_loop(0, nchunks // NB, ring, 0)

        finish(NB - 1, 1, None)
        for p in range(2):
            pltpu.make_async_copy(sbuf[p], acc.at[tscs[0]], sem_s[p]).wait()
        plsc.subcore_barrier()

        out_base = c * n_acc + base
        for j in range(nfull):
            pltpu.sync_copy(acc.at[pl.ds(base + j * CH, CH)], sbuf[0])
            pltpu.sync_copy(sbuf[0], out_h.at[pl.ds(out_base + j * CH, CH)])
        if rem:
            pltpu.sync_copy(acc.at[pl.ds(base + nfull * CH, rem)],
                            sbuf[0].at[pl.ds(0, rem)])
            pltpu.sync_copy(sbuf[0].at[pl.ds(0, rem)],
                            out_h.at[pl.ds(out_base + nfull * CH, rem)])

    return k(xm_packed, s_all, t_all, edata)


def _tc_combine(p0, p1, wx, blk):
    n, d = wx.shape
    nblk = n // blk

    def body(p0_ref, p1_ref, wx_ref, out_ref):
        out_ref[...] = jnp.maximum(
            p0_ref[...] + p1_ref[...] + wx_ref[...], 0.0)

    return pl.pallas_call(
        body,
        grid=(nblk,),
        in_specs=[
            pl.BlockSpec((blk, d), lambda i: (i, 0)),
            pl.BlockSpec((blk, d), lambda i: (i, 0)),
            pl.BlockSpec((blk, d), lambda i: (i, 0)),
        ],
        out_specs=pl.BlockSpec((blk, d), lambda i: (i, 0)),
        out_shape=jax.ShapeDtypeStruct((n, d), jnp.float32),
    )(p0, p1, wx)


def kernel(x, lower_neighborhood_indices, lower_neighborhood_values,
           upper_neighborhood_indices, upper_neighborhood_values,
           W_lower, att_lower, W_upper, att_upper, W_lin):
    n, d = x.shape
    e = lower_neighborhood_values.shape[0]

    A_lower = jnp.zeros((d, 8), jnp.float32)
    A_lower = A_lower.at[:, 0].set(att_lower[:d]).at[:, 1].set(att_lower[d:])
    A_upper = jnp.zeros((d, 8), jnp.float32)
    A_upper = A_upper.at[:, 2].set(att_upper[:d]).at[:, 3].set(att_upper[d:])

    xm_lo, xm_up, wx, aux = _tc_prep(x, W_lower, W_upper, W_lin,
                                     A_lower, A_upper, blk=400)

    xm_all = jnp.concatenate([xm_lo, xm_up], axis=0)
    s_all = jnp.concatenate([aux[:, 0], aux[:, 2]])
    t_all = jnp.concatenate([aux[:, 1], aux[:, 3]])

    nchunks = -(-(-(-e // (NS * CH))) // 4) * 4
    e_pad = NS * CH * nchunks
    pad = e_pad - e

    def prep(idx, vals, node_off):
        tgt = jnp.concatenate([idx[0], jnp.zeros((pad,), jnp.int32)])
        src = jnp.concatenate([idx[1], jnp.zeros((pad,), jnp.int32)]) + node_off
        v = jax.lax.bitcast_convert_type(
            jnp.concatenate([vals, jnp.zeros((pad,), jnp.float32)]), jnp.int32)
        return jnp.stack([src, tgt, v], axis=1)

    ed_lo = prep(lower_neighborhood_indices, lower_neighborhood_values, 0)
    ed_up = prep(upper_neighborhood_indices, upper_neighborhood_values, n)
    edata = (jnp.concatenate([ed_lo, ed_up])
             .reshape(NC * NS * nchunks, CH, 3)
             .transpose(0, 2, 1))

    perm = np.arange(128).reshape(4, 2, 16).transpose(0, 2, 1).reshape(-1)
    xm_b = xm_all.astype(jnp.bfloat16)[:, perm]
    xm_packed = jax.lax.bitcast_convert_type(
        xm_b.reshape(NC * n, 64, 2), jnp.int32)

    partials = _sc_message_pass(xm_packed, s_all, t_all, edata, n, d, nchunks)
    n_acc = partials.shape[0] // NC
    p0 = lax.slice(partials, (0, 0), (n, d))
    p1 = lax.slice(partials, (n_acc, 0), (n_acc + n, d))

    return _tc_combine(p0, p1, wx, blk=400)


# speedup vs baseline: 1.1450x; 1.1450x over previous
import functools

import jax
import jax.numpy as jnp
import numpy as np
from jax import lax
from jax.experimental import pallas as pl
from jax.experimental.pallas import tpu as pltpu
from jax.experimental.pallas import tpu_sc as plsc

EPS = 1.0 + 1e-06

NC = 2
NS = 16
CH = 64
NB = 3
UNROLL = 2


def _tc_prep(x, W_lower, W_upper, W_lin, A_lower, A_upper, blk):
    n, d = x.shape
    nblk = n // blk

    def body(x_ref, wl_ref, wu_ref, wn_ref, al_ref, au_ref,
             xml_ref, xmu_ref, wx_ref, aux_ref):
        xb = x_ref[...]
        xml = jnp.dot(xb, wl_ref[...], preferred_element_type=jnp.float32)
        xmu = jnp.dot(xb, wu_ref[...], preferred_element_type=jnp.float32)
        xml_ref[...] = xml
        xmu_ref[...] = xmu
        wx_ref[...] = jnp.dot(xb, wn_ref[...],
                              preferred_element_type=jnp.float32) * EPS
        aux_ref[...] = (jnp.dot(xml, al_ref[...],
                                preferred_element_type=jnp.float32) +
                        jnp.dot(xmu, au_ref[...],
                                preferred_element_type=jnp.float32))

    return pl.pallas_call(
        body,
        grid=(nblk,),
        in_specs=[
            pl.BlockSpec((blk, d), lambda i: (i, 0)),
            pl.BlockSpec((d, d), lambda i: (0, 0)),
            pl.BlockSpec((d, d), lambda i: (0, 0)),
            pl.BlockSpec((d, d), lambda i: (0, 0)),
            pl.BlockSpec((d, 8), lambda i: (0, 0)),
            pl.BlockSpec((d, 8), lambda i: (0, 0)),
        ],
        out_specs=[
            pl.BlockSpec((blk, d), lambda i: (i, 0)),
            pl.BlockSpec((blk, d), lambda i: (i, 0)),
            pl.BlockSpec((blk, d), lambda i: (i, 0)),
            pl.BlockSpec((blk, 8), lambda i: (i, 0)),
        ],
        out_shape=[
            jax.ShapeDtypeStruct((n, d), jnp.float32),
            jax.ShapeDtypeStruct((n, d), jnp.float32),
            jax.ShapeDtypeStruct((n, d), jnp.float32),
            jax.ShapeDtypeStruct((n, 8), jnp.float32),
        ],
    )(x, W_lower, W_upper, W_lin, A_lower, A_upper)


def _sc_message_pass(xm_all, s_all, t_all, edata, n, d, nchunks):
    ept = nchunks * CH
    rpt = -(-n // (NS * 8)) * 8
    n_acc = NS * rpt
    nfull = rpt // CH
    rem = rpt - nfull * CH

    mesh = plsc.VectorSubcoreMesh(core_axis_name="c", subcore_axis_name="s",
                                  num_cores=NC, num_subcores=NS)

    scratch = (
        [pltpu.VMEM((n,), jnp.float32)] * 2 +
        [pltpu.VMEM((3, CH), jnp.int32)] * NB +
        [pltpu.VMEM((CH,), jnp.int32)] * NB +
        [pltpu.VMEM((CH,), jnp.int32)] * NB +
        [pltpu.VMEM((CH,), jnp.float32)] * NB +
        [pltpu.VMEM((CH, 128), jnp.float32)] * NB +
        [pltpu.VMEM_SHARED((n_acc, 128), jnp.float32)] +
        [pltpu.SemaphoreType.DMA] * (3 * NB)
    )

    @functools.partial(
        pl.kernel,
        out_type=jax.ShapeDtypeStruct((NC * n_acc, d), jnp.float32),
        mesh=mesh,
        compiler_params=pltpu.CompilerParams(needs_layout_passes=False),
        scratch_types=scratch,
    )
    def k(xm_h, s_h, t_h, ed_h, out_h, s_v, t_v, *bufs):
        ebs = bufs[0:NB]
        tscs = bufs[NB:2 * NB]
        gscs = bufs[2 * NB:3 * NB]
        ews = bufs[3 * NB:4 * NB]
        rows = bufs[4 * NB:5 * NB]
        acc = bufs[5 * NB]
        sem_e = bufs[5 * NB + 1:5 * NB + 1 + NB]
        sem_r = bufs[5 * NB + 1 + NB:5 * NB + 1 + 2 * NB]
        sem_s = bufs[5 * NB + 1 + 2 * NB:5 * NB + 1 + 3 * NB]
        c = lax.axis_index("c")
        sid = lax.axis_index("s")
        base = sid * rpt

        pltpu.sync_copy(s_h.at[pl.ds(c * n, n)], s_v)
        pltpu.sync_copy(t_h.at[pl.ds(c * n, n)], t_v)

        def zero_rows(i, _):
            for kk in range(8):
                rows[0][i, pl.ds(kk * 16, 16)] = jnp.zeros((16,), jnp.float32)
            return 0
        lax.fori_loop(0, CH, zero_rows, 0)
        for j in range(nfull):
            pltpu.sync_copy(rows[0], acc.at[pl.ds(base + j * CH, CH)])
        if rem:
            pltpu.sync_copy(rows[0].at[pl.ds(0, rem)],
                            acc.at[pl.ds(base + nfull * CH, rem)])
        plsc.subcore_barrier()

        chunk_base = (c * NS + sid) * nchunks

        def issue_edata(j, b):
            pltpu.async_copy(ed_h.at[chunk_base + j], ebs[b], sem_e[b])

        def scale_and_scatter(bp):
            rowp = rows[bp]
            ewp = ews[bp]

            @plsc.parallel_loop(0, CH, unroll=UNROLL)
            def _(e2):
                ew16 = plsc.load_gather(ewp, [lax.broadcast(e2, (16,))])
                for kk in range(8):
                    slk = pl.ds(kk * 16, 16)
                    rowp[e2, slk] = rowp[e2, slk] * ew16
            pltpu.async_copy(rowp, acc.at[tscs[bp]], sem_s[bp], add=True)

        def wait_gather(bp):
            pltpu.make_async_copy(xm_h.at[gscs[bp]], rows[bp], sem_r[bp]).wait()

        def process(i, b):
            j = NB * i + b
            eb = ebs[b]
            tsc = tscs[b]
            gsc = gscs[b]
            ew_v = ews[b]
            row = rows[b]
            pltpu.make_async_copy(ed_h.at[chunk_base], eb, sem_e[b]).wait()
            @pl.when(i >= 1)
            def _():
                pltpu.make_async_copy(row, acc.at[tsc], sem_s[b]).wait()
            for i16 in range(CH // 16):
                sl = pl.ds(i16 * 16, 16)
                sv = eb[0, sl]
                tv = eb[1, sl]
                a = (plsc.load_gather(s_v, [sv - c * n]) +
                     plsc.load_gather(t_v, [tv]))
                att = jnp.where(a > 0.0, a, jnp.exp(a) - 1.0)
                ew_v[sl] = att * plsc.bitcast(eb[2, sl], jnp.float32)
                tsc[sl] = tv
                gsc[sl] = sv
            pltpu.async_copy(xm_h.at[gsc], row, sem_r[b])
            @pl.when(j + NB < nchunks)
            def _():
                issue_edata(j + NB, b)
            if b == 0:
                @pl.when(i >= 1)
                def _():
                    wait_gather(NB - 1)
                    scale_and_scatter(NB - 1)
            else:
                wait_gather(b - 1)
                scale_and_scatter(b - 1)

        for b in range(NB):
            issue_edata(b, b)

        def ring(i, _):
            for b in range(NB):
                process(i, b)
            return 0
        lax.fori_loop(0, nchunks // NB, ring, 0)

        wait_gather(NB - 1)
        scale_and_scatter(NB - 1)
        for b in range(NB):
            pltpu.make_async_copy(rows[b], acc.at[tscs[b]], sem_s[b]).wait()
        plsc.subcore_barrier()

        out_base = c * n_acc + base
        for j in range(nfull):
            pltpu.sync_copy(acc.at[pl.ds(base + j * CH, CH)], rows[0])
            pltpu.sync_copy(rows[0], out_h.at[pl.ds(out_base + j * CH, CH)])
        if rem:
            pltpu.sync_copy(acc.at[pl.ds(base + nfull * CH, rem)],
                            rows[0].at[pl.ds(0, rem)])
            pltpu.sync_copy(rows[0].at[pl.ds(0, rem)],
                            out_h.at[pl.ds(out_base + nfull * CH, rem)])

    return k(xm_all, s_all, t_all, edata)


def _tc_combine(p0, p1, wx, blk):
    n, d = wx.shape
    nblk = n // blk

    def body(p0_ref, p1_ref, wx_ref, out_ref):
        out_ref[...] = jnp.maximum(
            p0_ref[...] + p1_ref[...] + wx_ref[...], 0.0)

    return pl.pallas_call(
        body,
        grid=(nblk,),
        in_specs=[
            pl.BlockSpec((blk, d), lambda i: (i, 0)),
            pl.BlockSpec((blk, d), lambda i: (i, 0)),
            pl.BlockSpec((blk, d), lambda i: (i, 0)),
        ],
        out_specs=pl.BlockSpec((blk, d), lambda i: (i, 0)),
        out_shape=jax.ShapeDtypeStruct((n, d), jnp.float32),
    )(p0, p1, wx)


def kernel(x, lower_neighborhood_indices, lower_neighborhood_values,
           upper_neighborhood_indices, upper_neighborhood_values,
           W_lower, att_lower, W_upper, att_upper, W_lin):
    n, d = x.shape
    e = lower_neighborhood_values.shape[0]

    A_lower = jnp.zeros((d, 8), jnp.float32)
    A_lower = A_lower.at[:, 0].set(att_lower[:d]).at[:, 1].set(att_lower[d:])
    A_upper = jnp.zeros((d, 8), jnp.float32)
    A_upper = A_upper.at[:, 2].set(att_upper[:d]).at[:, 3].set(att_upper[d:])

    xm_lo, xm_up, wx, aux = _tc_prep(x, W_lower, W_upper, W_lin,
                                     A_lower, A_upper, blk=400)

    xm_all = jnp.concatenate([xm_lo, xm_up], axis=0)
    s_all = jnp.concatenate([aux[:, 0], aux[:, 2]])
    t_all = jnp.concatenate([aux[:, 1], aux[:, 3]])

    nchunks = -(-(-(-e // (NS * CH))) // NB) * NB
    e_pad = NS * CH * nchunks
    pad = e_pad - e

    def prep(idx, vals, node_off):
        tgt = jnp.concatenate([idx[0], jnp.zeros((pad,), jnp.int32)])
        src = jnp.concatenate([idx[1], jnp.zeros((pad,), jnp.int32)]) + node_off
        v = jax.lax.bitcast_convert_type(
            jnp.concatenate([vals, jnp.zeros((pad,), jnp.float32)]), jnp.int32)
        return jnp.stack([src, tgt, v], axis=1)

    ed_lo = prep(lower_neighborhood_indices, lower_neighborhood_values, 0)
    ed_up = prep(upper_neighborhood_indices, upper_neighborhood_values, n)
    edata = (jnp.concatenate([ed_lo, ed_up])
             .reshape(NC * NS * nchunks, CH, 3)
             .transpose(0, 2, 1))

    partials = _sc_message_pass(xm_all, s_all, t_all, edata, n, d, nchunks)
    n_acc = partials.shape[0] // NC
    p0 = lax.slice(partials, (0, 0), (n, d))
    p1 = lax.slice(partials, (n_acc, 0), (n_acc + n, d))

    return _tc_combine(p0, p1, wx, blk=400)


# speedup vs baseline: 1.1492x; 1.0037x over previous
import functools

import jax
import jax.numpy as jnp
import numpy as np
from jax import lax
from jax.experimental import pallas as pl
from jax.experimental.pallas import tpu as pltpu
from jax.experimental.pallas import tpu_sc as plsc

EPS = 1.0 + 1e-06

NC = 2
NS = 16
CH = 64
NB = 3
UNROLL = 4


def _tc_prep(x, W_lower, W_upper, W_lin, A_lower, A_upper, blk):
    n, d = x.shape
    nblk = n // blk

    def body(x_ref, wl_ref, wu_ref, wn_ref, al_ref, au_ref,
             xml_ref, xmu_ref, wx_ref, aux_ref):
        xb = x_ref[...]
        xml = jnp.dot(xb, wl_ref[...], preferred_element_type=jnp.float32)
        xmu = jnp.dot(xb, wu_ref[...], preferred_element_type=jnp.float32)
        xml_ref[...] = xml
        xmu_ref[...] = xmu
        wx_ref[...] = jnp.dot(xb, wn_ref[...],
                              preferred_element_type=jnp.float32) * EPS
        aux_ref[...] = (jnp.dot(xml, al_ref[...],
                                preferred_element_type=jnp.float32) +
                        jnp.dot(xmu, au_ref[...],
                                preferred_element_type=jnp.float32))

    return pl.pallas_call(
        body,
        grid=(nblk,),
        in_specs=[
            pl.BlockSpec((blk, d), lambda i: (i, 0)),
            pl.BlockSpec((d, d), lambda i: (0, 0)),
            pl.BlockSpec((d, d), lambda i: (0, 0)),
            pl.BlockSpec((d, d), lambda i: (0, 0)),
            pl.BlockSpec((d, 8), lambda i: (0, 0)),
            pl.BlockSpec((d, 8), lambda i: (0, 0)),
        ],
        out_specs=[
            pl.BlockSpec((blk, d), lambda i: (i, 0)),
            pl.BlockSpec((blk, d), lambda i: (i, 0)),
            pl.BlockSpec((blk, d), lambda i: (i, 0)),
            pl.BlockSpec((blk, 8), lambda i: (i, 0)),
        ],
        out_shape=[
            jax.ShapeDtypeStruct((n, d), jnp.float32),
            jax.ShapeDtypeStruct((n, d), jnp.float32),
            jax.ShapeDtypeStruct((n, d), jnp.float32),
            jax.ShapeDtypeStruct((n, 8), jnp.float32),
        ],
    )(x, W_lower, W_upper, W_lin, A_lower, A_upper)


def _sc_message_pass(xm_all, s_all, t_all, edata, n, d, nchunks):
    ept = nchunks * CH
    rpt = -(-n // (NS * 8)) * 8
    n_acc = NS * rpt
    nfull = rpt // CH
    rem = rpt - nfull * CH

    mesh = plsc.VectorSubcoreMesh(core_axis_name="c", subcore_axis_name="s",
                                  num_cores=NC, num_subcores=NS)

    scratch = (
        [pltpu.VMEM((n,), jnp.float32)] * 2 +
        [pltpu.VMEM((3, CH), jnp.int32)] * NB +
        [pltpu.VMEM((CH,), jnp.int32)] * NB +
        [pltpu.VMEM((CH,), jnp.int32)] * NB +
        [pltpu.VMEM((CH,), jnp.float32)] * NB +
        [pltpu.VMEM((CH, 128), jnp.float32)] * NB +
        [pltpu.VMEM_SHARED((n_acc, 128), jnp.float32)] +
        [pltpu.SemaphoreType.DMA] * (3 * NB)
    )

    @functools.partial(
        pl.kernel,
        out_type=jax.ShapeDtypeStruct((NC * n_acc, d), jnp.float32),
        mesh=mesh,
        compiler_params=pltpu.CompilerParams(needs_layout_passes=False),
        scratch_types=scratch,
    )
    def k(xm_h, s_h, t_h, ed_h, out_h, s_v, t_v, *bufs):
        ebs = bufs[0:NB]
        tscs = bufs[NB:2 * NB]
        gscs = bufs[2 * NB:3 * NB]
        ews = bufs[3 * NB:4 * NB]
        rows = bufs[4 * NB:5 * NB]
        acc = bufs[5 * NB]
        sem_e = bufs[5 * NB + 1:5 * NB + 1 + NB]
        sem_r = bufs[5 * NB + 1 + NB:5 * NB + 1 + 2 * NB]
        sem_s = bufs[5 * NB + 1 + 2 * NB:5 * NB + 1 + 3 * NB]
        c = lax.axis_index("c")
        sid = lax.axis_index("s")
        base = sid * rpt

        pltpu.sync_copy(s_h.at[pl.ds(c * n, n)], s_v)
        pltpu.sync_copy(t_h.at[pl.ds(c * n, n)], t_v)

        def zero_rows(i, _):
            for kk in range(8):
                rows[0][i, pl.ds(kk * 16, 16)] = jnp.zeros((16,), jnp.float32)
            return 0
        lax.fori_loop(0, CH, zero_rows, 0)
        for j in range(nfull):
            pltpu.sync_copy(rows[0], acc.at[pl.ds(base + j * CH, CH)])
        if rem:
            pltpu.sync_copy(rows[0].at[pl.ds(0, rem)],
                            acc.at[pl.ds(base + nfull * CH, rem)])
        plsc.subcore_barrier()

        chunk_base = (c * NS + sid) * nchunks

        def issue_edata(j, b):
            pltpu.async_copy(ed_h.at[chunk_base + j], ebs[b], sem_e[b])

        def scale_and_scatter(bp):
            rowp = rows[bp]
            ewp = ews[bp]

            @plsc.parallel_loop(0, CH, unroll=UNROLL)
            def _(e2):
                ew16 = plsc.load_gather(ewp, [lax.broadcast(e2, (16,))])
                for kk in range(8):
                    slk = pl.ds(kk * 16, 16)
                    rowp[e2, slk] = rowp[e2, slk] * ew16
            pltpu.async_copy(rowp, acc.at[tscs[bp]], sem_s[bp], add=True)

        def wait_gather(bp):
            pltpu.make_async_copy(xm_h.at[gscs[bp]], rows[bp], sem_r[bp]).wait()

        def process(i, b):
            j = NB * i + b
            eb = ebs[b]
            tsc = tscs[b]
            gsc = gscs[b]
            ew_v = ews[b]
            row = rows[b]
            pltpu.make_async_copy(ed_h.at[chunk_base], eb, sem_e[b]).wait()
            @pl.when(i >= 1)
            def _():
                pltpu.make_async_copy(row, acc.at[tsc], sem_s[b]).wait()
            for i16 in range(CH // 16):
                sl = pl.ds(i16 * 16, 16)
                sv = eb[0, sl]
                tv = eb[1, sl]
                a = (plsc.load_gather(s_v, [sv - c * n]) +
                     plsc.load_gather(t_v, [tv]))
                att = jnp.where(a > 0.0, a, jnp.exp(a) - 1.0)
                ew_v[sl] = att * plsc.bitcast(eb[2, sl], jnp.float32)
                tsc[sl] = tv
                gsc[sl] = sv
            pltpu.async_copy(xm_h.at[gsc], row, sem_r[b])
            @pl.when(j + NB < nchunks)
            def _():
                issue_edata(j + NB, b)
            if b == 0:
                @pl.when(i >= 1)
                def _():
                    wait_gather(NB - 1)
                    scale_and_scatter(NB - 1)
            else:
                wait_gather(b - 1)
                scale_and_scatter(b - 1)

        for b in range(NB):
            issue_edata(b, b)

        def ring(i, _):
            for b in range(NB):
                process(i, b)
            return 0
        lax.fori_loop(0, nchunks // NB, ring, 0)

        wait_gather(NB - 1)
        scale_and_scatter(NB - 1)
        for b in range(NB):
            pltpu.make_async_copy(rows[b], acc.at[tscs[b]], sem_s[b]).wait()
        plsc.subcore_barrier()

        out_base = c * n_acc + base
        for j in range(nfull):
            pltpu.sync_copy(acc.at[pl.ds(base + j * CH, CH)], rows[0])
            pltpu.sync_copy(rows[0], out_h.at[pl.ds(out_base + j * CH, CH)])
        if rem:
            pltpu.sync_copy(acc.at[pl.ds(base + nfull * CH, rem)],
                            rows[0].at[pl.ds(0, rem)])
            pltpu.sync_copy(rows[0].at[pl.ds(0, rem)],
                            out_h.at[pl.ds(out_base + nfull * CH, rem)])

    return k(xm_all, s_all, t_all, edata)


def _tc_combine(p0, p1, wx, blk):
    n, d = wx.shape
    nblk = n // blk

    def body(p0_ref, p1_ref, wx_ref, out_ref):
        out_ref[...] = jnp.maximum(
            p0_ref[...] + p1_ref[...] + wx_ref[...], 0.0)

    return pl.pallas_call(
        body,
        grid=(nblk,),
        in_specs=[
            pl.BlockSpec((blk, d), lambda i: (i, 0)),
            pl.BlockSpec((blk, d), lambda i: (i, 0)),
            pl.BlockSpec((blk, d), lambda i: (i, 0)),
        ],
        out_specs=pl.BlockSpec((blk, d), lambda i: (i, 0)),
        out_shape=jax.ShapeDtypeStruct((n, d), jnp.float32),
    )(p0, p1, wx)


def kernel(x, lower_neighborhood_indices, lower_neighborhood_values,
           upper_neighborhood_indices, upper_neighborhood_values,
           W_lower, att_lower, W_upper, att_upper, W_lin):
    n, d = x.shape
    e = lower_neighborhood_values.shape[0]

    A_lower = jnp.zeros((d, 8), jnp.float32)
    A_lower = A_lower.at[:, 0].set(att_lower[:d]).at[:, 1].set(att_lower[d:])
    A_upper = jnp.zeros((d, 8), jnp.float32)
    A_upper = A_upper.at[:, 2].set(att_upper[:d]).at[:, 3].set(att_upper[d:])

    xm_lo, xm_up, wx, aux = _tc_prep(x, W_lower, W_upper, W_lin,
                                     A_lower, A_upper, blk=400)

    xm_all = jnp.concatenate([xm_lo, xm_up], axis=0)
    s_all = jnp.concatenate([aux[:, 0], aux[:, 2]])
    t_all = jnp.concatenate([aux[:, 1], aux[:, 3]])

    nchunks = -(-(-(-e // (NS * CH))) // NB) * NB
    e_pad = NS * CH * nchunks
    pad = e_pad - e

    def prep(idx, vals, node_off):
        tgt = jnp.concatenate([idx[0], jnp.zeros((pad,), jnp.int32)])
        src = jnp.concatenate([idx[1], jnp.zeros((pad,), jnp.int32)]) + node_off
        v = jax.lax.bitcast_convert_type(
            jnp.concatenate([vals, jnp.zeros((pad,), jnp.float32)]), jnp.int32)
        return jnp.stack([src, tgt, v], axis=1)

    ed_lo = prep(lower_neighborhood_indices, lower_neighborhood_values, 0)
    ed_up = prep(upper_neighborhood_indices, upper_neighborhood_values, n)
    edata = (jnp.concatenate([ed_lo, ed_up])
             .reshape(NC * NS * nchunks, CH, 3)
             .transpose(0, 2, 1))

    partials = _sc_message_pass(xm_all, s_all, t_all, edata, n, d, nchunks)
    n_acc = partials.shape[0] // NC
    p0 = lax.slice(partials, (0, 0), (n, d))
    p1 = lax.slice(partials, (n_acc, 0), (n_acc + n, d))

    return _tc_combine(p0, p1, wx, blk=400)


# speedup vs baseline: 1.1497x; 1.0004x over previous
import functools

import jax
import jax.numpy as jnp
import numpy as np
from jax import lax
from jax.experimental import pallas as pl
from jax.experimental.pallas import tpu as pltpu
from jax.experimental.pallas import tpu_sc as plsc

EPS = 1.0 + 1e-06

NC = 2
NS = 16
CH = 64
NB = 3
UNROLL = 4


def _tc_prep(x, W_lower, W_upper, W_lin, A_lower, A_upper, blk):
    n, d = x.shape
    nblk = n // blk

    def body(x_ref, wl_ref, wu_ref, wn_ref, al_ref, au_ref,
             xm_ref, wx_ref, aux_ref):
        xb = x_ref[...]
        xml = jnp.dot(xb, wl_ref[...], preferred_element_type=jnp.float32)
        xmu = jnp.dot(xb, wu_ref[...], preferred_element_type=jnp.float32)
        xm_ref[0] = xml
        xm_ref[1] = xmu
        wx_ref[...] = jnp.dot(xb, wn_ref[...],
                              preferred_element_type=jnp.float32) * EPS
        aux_ref[...] = (jnp.dot(xml, al_ref[...],
                                preferred_element_type=jnp.float32) +
                        jnp.dot(xmu, au_ref[...],
                                preferred_element_type=jnp.float32))

    return pl.pallas_call(
        body,
        grid=(nblk,),
        in_specs=[
            pl.BlockSpec((blk, d), lambda i: (i, 0)),
            pl.BlockSpec((d, d), lambda i: (0, 0)),
            pl.BlockSpec((d, d), lambda i: (0, 0)),
            pl.BlockSpec((d, d), lambda i: (0, 0)),
            pl.BlockSpec((d, 8), lambda i: (0, 0)),
            pl.BlockSpec((d, 8), lambda i: (0, 0)),
        ],
        out_specs=[
            pl.BlockSpec((2, blk, d), lambda i: (0, i, 0)),
            pl.BlockSpec((blk, d), lambda i: (i, 0)),
            pl.BlockSpec((blk, 8), lambda i: (i, 0)),
        ],
        out_shape=[
            jax.ShapeDtypeStruct((2, n, d), jnp.float32),
            jax.ShapeDtypeStruct((n, d), jnp.float32),
            jax.ShapeDtypeStruct((n, 8), jnp.float32),
        ],
    )(x, W_lower, W_upper, W_lin, A_lower, A_upper)


def _sc_message_pass(xm_all, s_all, t_all, edata, n, d, nchunks):
    ept = nchunks * CH
    rpt = -(-n // (NS * 8)) * 8
    n_acc = NS * rpt
    nfull = rpt // CH
    rem = rpt - nfull * CH

    mesh = plsc.VectorSubcoreMesh(core_axis_name="c", subcore_axis_name="s",
                                  num_cores=NC, num_subcores=NS)

    scratch = (
        [pltpu.VMEM((n,), jnp.float32)] * 2 +
        [pltpu.VMEM((3, CH), jnp.int32)] * NB +
        [pltpu.VMEM((CH,), jnp.int32)] * NB +
        [pltpu.VMEM((CH,), jnp.int32)] * NB +
        [pltpu.VMEM((CH,), jnp.float32)] * NB +
        [pltpu.VMEM((CH, 128), jnp.float32)] * NB +
        [pltpu.VMEM_SHARED((n_acc, 128), jnp.float32)] +
        [pltpu.SemaphoreType.DMA] * (3 * NB)
    )

    @functools.partial(
        pl.kernel,
        out_type=[jax.ShapeDtypeStruct((n_acc, d), jnp.float32),
                  jax.ShapeDtypeStruct((n_acc, d), jnp.float32)],
        mesh=mesh,
        compiler_params=pltpu.CompilerParams(needs_layout_passes=False),
        scratch_types=scratch,
    )
    def k(xm_h, s_h, t_h, ed_h, out0_h, out1_h, s_v, t_v, *bufs):
        ebs = bufs[0:NB]
        tscs = bufs[NB:2 * NB]
        gscs = bufs[2 * NB:3 * NB]
        ews = bufs[3 * NB:4 * NB]
        rows = bufs[4 * NB:5 * NB]
        acc = bufs[5 * NB]
        sem_e = bufs[5 * NB + 1:5 * NB + 1 + NB]
        sem_r = bufs[5 * NB + 1 + NB:5 * NB + 1 + 2 * NB]
        sem_s = bufs[5 * NB + 1 + 2 * NB:5 * NB + 1 + 3 * NB]
        c = lax.axis_index("c")
        sid = lax.axis_index("s")
        base = sid * rpt

        pltpu.sync_copy(s_h.at[pl.ds(c * n, n)], s_v)
        pltpu.sync_copy(t_h.at[pl.ds(c * n, n)], t_v)

        def zero_rows(i, _):
            for kk in range(8):
                rows[0][i, pl.ds(kk * 16, 16)] = jnp.zeros((16,), jnp.float32)
            return 0
        lax.fori_loop(0, CH, zero_rows, 0)
        for j in range(nfull):
            pltpu.sync_copy(rows[0], acc.at[pl.ds(base + j * CH, CH)])
        if rem:
            pltpu.sync_copy(rows[0].at[pl.ds(0, rem)],
                            acc.at[pl.ds(base + nfull * CH, rem)])
        plsc.subcore_barrier()

        chunk_base = (c * NS + sid) * nchunks

        def issue_edata(j, b):
            pltpu.async_copy(ed_h.at[chunk_base + j], ebs[b], sem_e[b])

        def scale_and_scatter(bp):
            rowp = rows[bp]
            ewp = ews[bp]

            @plsc.parallel_loop(0, CH, unroll=UNROLL)
            def _(e2):
                ew16 = plsc.load_gather(ewp, [lax.broadcast(e2, (16,))])
                for kk in range(8):
                    slk = pl.ds(kk * 16, 16)
                    rowp[e2, slk] = rowp[e2, slk] * ew16
            pltpu.async_copy(rowp, acc.at[tscs[bp]], sem_s[bp], add=True)

        def wait_gather(bp):
            pltpu.make_async_copy(xm_h.at[gscs[bp]], rows[bp], sem_r[bp]).wait()

        def process(i, b):
            j = NB * i + b
            eb = ebs[b]
            tsc = tscs[b]
            gsc = gscs[b]
            ew_v = ews[b]
            row = rows[b]
            pltpu.make_async_copy(ed_h.at[chunk_base], eb, sem_e[b]).wait()
            @pl.when(i >= 1)
            def _():
                pltpu.make_async_copy(row, acc.at[tsc], sem_s[b]).wait()
            for i16 in range(CH // 16):
                sl = pl.ds(i16 * 16, 16)
                sv = eb[0, sl]
                tv = eb[1, sl]
                a = (plsc.load_gather(s_v, [sv - c * n]) +
                     plsc.load_gather(t_v, [tv]))
                att = jnp.where(a > 0.0, a, jnp.exp(a) - 1.0)
                ew_v[sl] = att * plsc.bitcast(eb[2, sl], jnp.float32)
                tsc[sl] = tv
                gsc[sl] = sv
            pltpu.async_copy(xm_h.at[gsc], row, sem_r[b])
            @pl.when(j + NB < nchunks)
            def _():
                issue_edata(j + NB, b)
            if b == 0:
                @pl.when(i >= 1)
                def _():
                    wait_gather(NB - 1)
                    scale_and_scatter(NB - 1)
            else:
                wait_gather(b - 1)
                scale_and_scatter(b - 1)

        for b in range(NB):
            issue_edata(b, b)

        def ring(i, _):
            for b in range(NB):
                process(i, b)
            return 0
        lax.fori_loop(0, nchunks // NB, ring, 0)

        wait_gather(NB - 1)
        scale_and_scatter(NB - 1)
        for b in range(NB):
            pltpu.make_async_copy(rows[b], acc.at[tscs[b]], sem_s[b]).wait()
        plsc.subcore_barrier()

        def writeout(out_h):
            for j in range(nfull):
                pltpu.sync_copy(acc.at[pl.ds(base + j * CH, CH)], rows[0])
                pltpu.sync_copy(rows[0], out_h.at[pl.ds(base + j * CH, CH)])
            if rem:
                pltpu.sync_copy(acc.at[pl.ds(base + nfull * CH, rem)],
                                rows[0].at[pl.ds(0, rem)])
                pltpu.sync_copy(rows[0].at[pl.ds(0, rem)],
                                out_h.at[pl.ds(base + nfull * CH, rem)])

        @pl.when(c == 0)
        def _():
            writeout(out0_h)

        @pl.when(c == 1)
        def _():
            writeout(out1_h)

    return k(xm_all, s_all, t_all, edata)


def _tc_combine(p0, p1, wx, blk):
    n, d = wx.shape
    nblk = n // blk

    def body(p0_ref, p1_ref, wx_ref, out_ref):
        out_ref[...] = jnp.maximum(
            p0_ref[...] + p1_ref[...] + wx_ref[...], 0.0)

    return pl.pallas_call(
        body,
        grid=(nblk,),
        in_specs=[
            pl.BlockSpec((blk, d), lambda i: (i, 0)),
            pl.BlockSpec((blk, d), lambda i: (i, 0)),
            pl.BlockSpec((blk, d), lambda i: (i, 0)),
        ],
        out_specs=pl.BlockSpec((blk, d), lambda i: (i, 0)),
        out_shape=jax.ShapeDtypeStruct((n, d), jnp.float32),
    )(p0, p1, wx)


def kernel(x, lower_neighborhood_indices, lower_neighborhood_values,
           upper_neighborhood_indices, upper_neighborhood_values,
           W_lower, att_lower, W_upper, att_upper, W_lin):
    n, d = x.shape
    e = lower_neighborhood_values.shape[0]

    A_lower = jnp.zeros((d, 8), jnp.float32)
    A_lower = A_lower.at[:, 0].set(att_lower[:d]).at[:, 1].set(att_lower[d:])
    A_upper = jnp.zeros((d, 8), jnp.float32)
    A_upper = A_upper.at[:, 2].set(att_upper[:d]).at[:, 3].set(att_upper[d:])

    xm3, wx, aux = _tc_prep(x, W_lower, W_upper, W_lin,
                            A_lower, A_upper, blk=400)

    xm_all = xm3.reshape(NC * n, d)
    s_all = jnp.concatenate([aux[:, 0], aux[:, 2]])
    t_all = jnp.concatenate([aux[:, 1], aux[:, 3]])

    nchunks = -(-(-(-e // (NS * CH))) // NB) * NB
    e_pad = NS * CH * nchunks
    pad = e_pad - e

    def prep(idx, vals, node_off):
        tgt = jnp.concatenate([idx[0], jnp.zeros((pad,), jnp.int32)])
        src = jnp.concatenate([idx[1], jnp.zeros((pad,), jnp.int32)]) + node_off
        v = jax.lax.bitcast_convert_type(
            jnp.concatenate([vals, jnp.zeros((pad,), jnp.float32)]), jnp.int32)
        return jnp.stack([src.reshape(NS * nchunks, CH),
                          tgt.reshape(NS * nchunks, CH),
                          v.reshape(NS * nchunks, CH)], axis=1)

    ed_lo = prep(lower_neighborhood_indices, lower_neighborhood_values, 0)
    ed_up = prep(upper_neighborhood_indices, upper_neighborhood_values, n)
    edata = jnp.concatenate([ed_lo, ed_up])

    p0, p1 = _sc_message_pass(xm_all, s_all, t_all, edata, n, d, nchunks)

    return _tc_combine(p0, p1, wx, blk=400)
